# baseline (speedup 1.0000x reference)
import functools

import jax
import jax.numpy as jnp
from jax import lax
from jax.experimental import pallas as pl
from jax.experimental.pallas import tpu as pltpu
from jax.experimental.pallas import tpu_sc as plsc

K = 6
N_COARSE = 2048
N_FINE = 8192
D_IN = 256
D_OUT = 256
BF = 1024

NW = 32
ROWS_PER_W = N_FINE // NW
CHUNK = 16
NCHUNK = ROWS_PER_W // CHUNK


def _proj_kernel(x_ref, w_ref, o_ref):
    o_ref[:, :] = jnp.dot(x_ref[:, :], w_ref[:, :],
                          preferred_element_type=jnp.float32)


def _knn_kernel(pf_ref, pcT_ref, iota_ref, oi_ref):
    d = jnp.zeros((BF, N_COARSE), jnp.float32)
    for c in range(3):
        diff = pf_ref[:, c:c + 1] - pcT_ref[c:c + 1, :]
        d = d + diff * diff

    cols = []
    for _ in range(K):
        m = jnp.min(d, axis=1, keepdims=True)
        hit = d == m
        sel = jnp.where(hit, 1.0, 0.0)
        d = jnp.where(hit, jnp.inf, d)
        digits = jnp.dot(sel, iota_ref[:, :], preferred_element_type=jnp.float32)
        cols.append(digits[:, 0:1] * 64.0 + digits[:, 1:2])
    idx = (jnp.concatenate(cols, axis=1) + 0.5).astype(jnp.int32)
    oi_ref[:, :] = jnp.minimum(idx, N_COARSE - 1)


def _sc_gather_kernel(y_hbm, idx_hbm, b_hbm, out_hbm,
                      idx_v, rows_v, out_v, b_v, sem):
    wid = lax.axis_index("s") * 2 + lax.axis_index("c")
    base = wid * ROWS_PER_W
    pltpu.sync_copy(b_hbm, b_v)

    @pl.loop(0, NCHUNK, unroll=1)
    def _chunk(ch):
        row0 = base + ch * CHUNK
        pltpu.sync_copy(idx_hbm.at[pl.ds(row0 * K, CHUNK * K)], idx_v)
        pltpu.async_copy(y_hbm.at[idx_v], rows_v, sem).wait()

        @pl.loop(0, CHUNK, unroll=1)
        def _row(r):
            for cb in range(D_OUT // 16):
                col = pl.ds(cb * 16, 16)
                s = rows_v[r * K + 0, col]
                for t in range(1, K):
                    s = s + rows_v[r * K + t, col]
                out_v[r, col] = s * (1.0 / K) + b_v[col]

        pltpu.sync_copy(out_v, out_hbm.at[pl.ds(row0, CHUNK)])


@jax.jit
def kernel(x_coarse, pos_coarse, pos_fine, W, b):
    y = pl.pallas_call(
        _proj_kernel,
        out_shape=jax.ShapeDtypeStruct((N_COARSE, D_OUT), jnp.float32),
    )(x_coarse, W)

    pcT = pos_coarse.T
    ii = lax.broadcasted_iota(jnp.float32, (N_COARSE, 1), 0)
    iota_col = jnp.concatenate([jnp.floor(ii / 64.0), jnp.mod(ii, 64.0)],
                               axis=1)

    grid = N_FINE // BF
    knn_idx = pl.pallas_call(
        _knn_kernel,
        grid=(grid,),
        in_specs=[
            pl.BlockSpec((BF, 3), lambda i: (i, 0)),
            pl.BlockSpec((3, N_COARSE), lambda i: (0, 0)),
            pl.BlockSpec((N_COARSE, 2), lambda i: (0, 0)),
        ],
        out_specs=pl.BlockSpec((BF, K), lambda i: (i, 0)),
        out_shape=jax.ShapeDtypeStruct((N_FINE, K), jnp.int32),
    )(pos_fine, pcT, iota_col)

    idx_flat = knn_idx.reshape(N_FINE * K)

    sc = pl.kernel(
        _sc_gather_kernel,
        mesh=plsc.VectorSubcoreMesh(core_axis_name="c", subcore_axis_name="s"),
        out_type=jax.ShapeDtypeStruct((N_FINE, D_OUT), jnp.float32),
        scratch_types=[
            pltpu.VMEM((CHUNK * K,), jnp.int32),
            pltpu.VMEM((CHUNK * K, D_OUT), jnp.float32),
            pltpu.VMEM((CHUNK, D_OUT), jnp.float32),
            pltpu.VMEM((D_OUT,), jnp.float32),
            pltpu.SemaphoreType.DMA,
        ],
    )
    out = sc(y, idx_flat, b)
    return out


# speedup vs baseline: 1.1124x; 1.1124x over previous
import functools

import jax
import jax.numpy as jnp
from jax import lax
from jax.experimental import pallas as pl
from jax.experimental.pallas import tpu as pltpu
from jax.experimental.pallas import tpu_sc as plsc

K = 6
N_COARSE = 2048
N_FINE = 8192
D_IN = 256
D_OUT = 256
BF = 1024

NW = 32
ROWS_PER_W = N_FINE // NW
CHUNK = 16
NCHUNK = ROWS_PER_W // CHUNK


def _proj_kernel(x_ref, w_ref, o_ref):
    o_ref[:, :] = jnp.dot(x_ref[:, :], w_ref[:, :],
                          preferred_element_type=jnp.float32)


def _knn_kernel(pf_ref, pcT_ref, iota_ref, oi_ref):
    d = jnp.zeros((BF, N_COARSE), jnp.float32)
    for c in range(3):
        diff = pf_ref[:, c:c + 1] - pcT_ref[c:c + 1, :]
        d = d + diff * diff

    cols = []
    for _ in range(K):
        m = jnp.min(d, axis=1, keepdims=True)
        hit = d == m
        sel = jnp.where(hit, 1.0, 0.0)
        d = jnp.where(hit, jnp.inf, d)
        digits = jnp.dot(sel, iota_ref[:, :], preferred_element_type=jnp.float32)
        cols.append(digits[:, 0:1] * 64.0 + digits[:, 1:2])
    idx = (jnp.concatenate(cols, axis=1) + 0.5).astype(jnp.int32)
    oi_ref[:, :] = jnp.minimum(idx, N_COARSE - 1)


def _sc_gather_kernel(y_hbm, idx_hbm, b_hbm, out_hbm,
                      idx_v0, idx_v1, rows_v0, rows_v1, out_v, b_v,
                      sem0, sem1):
    wid = lax.axis_index("s") * 2 + lax.axis_index("c")
    base = wid * ROWS_PER_W
    pltpu.sync_copy(b_hbm, b_v)

    idx_bufs = (idx_v0, idx_v1)
    row_bufs = (rows_v0, rows_v1)
    sems = (sem0, sem1)

    def start_gather(ch, bi):
        pltpu.sync_copy(
            idx_hbm.at[pl.ds((base + ch * CHUNK) * K, CHUNK * K)],
            idx_bufs[bi])
        pltpu.async_copy(y_hbm.at[idx_bufs[bi]], row_bufs[bi], sems[bi])

    start_gather(0, 0)
    start_gather(1, 1)

    @pl.loop(0, NCHUNK // 2, unroll=1)
    def _pair(g):
        for b in range(2):
            ch = g * 2 + b
            rows_v = row_bufs[b]
            pltpu.make_async_copy(y_hbm.at[idx_bufs[b]], rows_v,
                                  sems[b]).wait()

            @pl.loop(0, CHUNK, unroll=1)
            def _row(r):
                for cb in range(D_OUT // 16):
                    col = pl.ds(cb * 16, 16)
                    s = rows_v[r * K + 0, col]
                    for t in range(1, K):
                        s = s + rows_v[r * K + t, col]
                    out_v[r, col] = s * (1.0 / K) + b_v[col]

            pltpu.sync_copy(out_v,
                            out_hbm.at[pl.ds(base + ch * CHUNK, CHUNK)])

            @pl.when(ch + 2 < NCHUNK)
            def _():
                start_gather(ch + 2, b)


@jax.jit
def kernel(x_coarse, pos_coarse, pos_fine, W, b):
    y = pl.pallas_call(
        _proj_kernel,
        out_shape=jax.ShapeDtypeStruct((N_COARSE, D_OUT), jnp.float32),
    )(x_coarse, W)

    pcT = pos_coarse.T
    ii = lax.broadcasted_iota(jnp.float32, (N_COARSE, 1), 0)
    iota_col = jnp.concatenate([jnp.floor(ii / 64.0), jnp.mod(ii, 64.0)],
                               axis=1)

    grid = N_FINE // BF
    knn_idx = pl.pallas_call(
        _knn_kernel,
        grid=(grid,),
        in_specs=[
            pl.BlockSpec((BF, 3), lambda i: (i, 0)),
            pl.BlockSpec((3, N_COARSE), lambda i: (0, 0)),
            pl.BlockSpec((N_COARSE, 2), lambda i: (0, 0)),
        ],
        out_specs=pl.BlockSpec((BF, K), lambda i: (i, 0)),
        out_shape=jax.ShapeDtypeStruct((N_FINE, K), jnp.int32),
    )(pos_fine, pcT, iota_col)

    idx_flat = knn_idx.reshape(N_FINE * K)

    sc = pl.kernel(
        _sc_gather_kernel,
        mesh=plsc.VectorSubcoreMesh(core_axis_name="c", subcore_axis_name="s"),
        out_type=jax.ShapeDtypeStruct((N_FINE, D_OUT), jnp.float32),
        scratch_types=[
            pltpu.VMEM((CHUNK * K,), jnp.int32),
            pltpu.VMEM((CHUNK * K,), jnp.int32),
            pltpu.VMEM((CHUNK * K, D_OUT), jnp.float32),
            pltpu.VMEM((CHUNK * K, D_OUT), jnp.float32),
            pltpu.VMEM((CHUNK, D_OUT), jnp.float32),
            pltpu.VMEM((D_OUT,), jnp.float32),
            pltpu.SemaphoreType.DMA,
            pltpu.SemaphoreType.DMA,
        ],
    )
    out = sc(y, idx_flat, b)
    return out


# speedup vs baseline: 1.1124x; 1.0000x over previous
import functools

import jax
import jax.numpy as jnp
from jax import lax
from jax.experimental import pallas as pl
from jax.experimental.pallas import tpu as pltpu
from jax.experimental.pallas import tpu_sc as plsc

K = 6
N_COARSE = 2048
N_FINE = 8192
D_IN = 256
D_OUT = 256
BF = 1024

NW = 32
ROWS_PER_W = N_FINE // NW
CHUNK = 16
NCHUNK = ROWS_PER_W // CHUNK


def _proj_kernel(x_ref, w_ref, o_ref):
    o_ref[:, :] = jnp.dot(x_ref[:, :], w_ref[:, :],
                          preferred_element_type=jnp.float32)


def _knn_kernel(pf_ref, pcT_ref, iota_ref, oi_ref):
    d = jnp.zeros((BF, N_COARSE), jnp.float32)
    for c in range(3):
        diff = pf_ref[:, c:c + 1] - pcT_ref[c:c + 1, :]
        d = d + diff * diff

    cols = []
    for _ in range(K):
        m = jnp.min(d, axis=1, keepdims=True)
        hit = d == m
        sel = jnp.where(hit, 1.0, 0.0)
        d = jnp.where(hit, jnp.inf, d)
        digits = jnp.dot(sel, iota_ref[:, :], preferred_element_type=jnp.float32)
        cols.append(digits[:, 0:1] * 64.0 + digits[:, 1:2])
    idx = (jnp.concatenate(cols, axis=1) + 0.5).astype(jnp.int32)
    oi_ref[:, :] = jnp.minimum(idx, N_COARSE - 1)


def _sc_gather_kernel(y_hbm, idx_hbm, b_hbm, out_hbm,
                      idx_v0, idx_v1, rows_v0, rows_v1, out_v, b_v,
                      sem0, sem1):
    wid = lax.axis_index("s") * 2 + lax.axis_index("c")
    base = wid * ROWS_PER_W
    pltpu.sync_copy(b_hbm, b_v)

    idx_bufs = (idx_v0, idx_v1)
    row_bufs = (rows_v0, rows_v1)
    sems = (sem0, sem1)

    def start_gather(ch, bi):
        pltpu.sync_copy(
            idx_hbm.at[pl.ds((base + ch * CHUNK) * K, CHUNK * K)],
            idx_bufs[bi])
        pltpu.async_copy(y_hbm.at[idx_bufs[bi]], row_bufs[bi], sems[bi])

    start_gather(0, 0)
    start_gather(1, 1)

    @pl.loop(0, NCHUNK // 2, unroll=1)
    def _pair(g):
        for b in range(2):
            ch = g * 2 + b
            rows_v = row_bufs[b]
            pltpu.make_async_copy(y_hbm.at[idx_bufs[b]], rows_v,
                                  sems[b]).wait()

            @pl.loop(0, CHUNK, unroll=2)
            def _row(r):
                for cb in range(D_OUT // 16):
                    col = pl.ds(cb * 16, 16)
                    s = rows_v[r * K + 0, col]
                    for t in range(1, K):
                        s = s + rows_v[r * K + t, col]
                    out_v[r, col] = s * (1.0 / K) + b_v[col]

            pltpu.sync_copy(out_v,
                            out_hbm.at[pl.ds(base + ch * CHUNK, CHUNK)])

            @pl.when(ch + 2 < NCHUNK)
            def _():
                start_gather(ch + 2, b)


@jax.jit
def kernel(x_coarse, pos_coarse, pos_fine, W, b):
    y = pl.pallas_call(
        _proj_kernel,
        out_shape=jax.ShapeDtypeStruct((N_COARSE, D_OUT), jnp.float32),
    )(x_coarse, W)

    pcT = pos_coarse.T
    ii = lax.broadcasted_iota(jnp.float32, (N_COARSE, 1), 0)
    iota_col = jnp.concatenate([jnp.floor(ii / 64.0), jnp.mod(ii, 64.0)],
                               axis=1)

    grid = N_FINE // BF
    knn_idx = pl.pallas_call(
        _knn_kernel,
        grid=(grid,),
        in_specs=[
            pl.BlockSpec((BF, 3), lambda i: (i, 0)),
            pl.BlockSpec((3, N_COARSE), lambda i: (0, 0)),
            pl.BlockSpec((N_COARSE, 2), lambda i: (0, 0)),
        ],
        out_specs=pl.BlockSpec((BF, K), lambda i: (i, 0)),
        out_shape=jax.ShapeDtypeStruct((N_FINE, K), jnp.int32),
    )(pos_fine, pcT, iota_col)

    idx_flat = knn_idx.reshape(N_FINE * K)

    sc = pl.kernel(
        _sc_gather_kernel,
        mesh=plsc.VectorSubcoreMesh(core_axis_name="c", subcore_axis_name="s"),
        out_type=jax.ShapeDtypeStruct((N_FINE, D_OUT), jnp.float32),
        scratch_types=[
            pltpu.VMEM((CHUNK * K,), jnp.int32),
            pltpu.VMEM((CHUNK * K,), jnp.int32),
            pltpu.VMEM((CHUNK * K, D_OUT), jnp.float32),
            pltpu.VMEM((CHUNK * K, D_OUT), jnp.float32),
            pltpu.VMEM((CHUNK, D_OUT), jnp.float32),
            pltpu.VMEM((D_OUT,), jnp.float32),
            pltpu.SemaphoreType.DMA,
            pltpu.SemaphoreType.DMA,
        ],
    )
    out = sc(y, idx_flat, b)
    return out
